# baseline (speedup 1.0000x reference)
import functools

import jax
import jax.numpy as jnp
from jax import lax
from jax.experimental import pallas as pl
from jax.experimental.pallas import tpu as pltpu
from jax.experimental.pallas import tpu_sc as plsc

_NUM_CORES = 2
_NUM_SUBCORES = 16
_NW = _NUM_CORES * _NUM_SUBCORES
_LANES = 16
_CHUNK = 128


@functools.cache
def _build(B, D):
    rows_per_w = B // _NW
    n_chunks = rows_per_w // _CHUNK
    vecs_per_row = D // _LANES
    assert rows_per_w * _NW == B and n_chunks * _CHUNK == rows_per_w
    assert vecs_per_row * _LANES == D

    mesh = plsc.VectorSubcoreMesh(core_axis_name="c", subcore_axis_name="s")

    @functools.partial(
        pl.kernel,
        out_type=jax.ShapeDtypeStruct((_NW * _LANES,), jnp.float32),
        mesh=mesh,
        scratch_types=[
            pltpu.VMEM((_CHUNK,), jnp.int32),
            pltpu.VMEM((_CHUNK, D), jnp.float32),
            pltpu.VMEM((_CHUNK, D), jnp.float32),
            pltpu.VMEM((_CHUNK, D), jnp.float32),
            pltpu.VMEM((_LANES,), jnp.float32),
            pltpu.SemaphoreType.DMA,
            pltpu.SemaphoreType.DMA,
            pltpu.SemaphoreType.DMA,
        ],
    )
    def sc_kernel(feat_hbm, a_hbm, label_hbm, centers_hbm, out_hbm,
                  idx_v, cent_v, feat_v, a_v, acc_v, sem_g, sem_f, sem_a):
        wid = lax.axis_index("s") * _NUM_CORES + lax.axis_index("c")
        base = wid * rows_per_w
        acc = tuple(jnp.zeros((_LANES,), jnp.float32)
                    for _ in range(vecs_per_row))
        for ci in range(n_chunks):
            row0 = base + ci * _CHUNK
            pltpu.sync_copy(label_hbm.at[pl.ds(row0, _CHUNK)], idx_v)
            cg = pltpu.async_copy(centers_hbm.at[idx_v], cent_v, sem_g)
            cf = pltpu.async_copy(feat_hbm.at[pl.ds(row0, _CHUNK), :],
                                  feat_v, sem_f)
            ca = pltpu.async_copy(a_hbm.at[pl.ds(row0, _CHUNK), :],
                                  a_v, sem_a)
            cg.wait()
            cf.wait()
            ca.wait()

            def row_body(r, accs):
                new = []
                for j in range(vecs_per_row):
                    f = feat_v[r, pl.ds(j * _LANES, _LANES)]
                    c = cent_v[r, pl.ds(j * _LANES, _LANES)]
                    w = a_v[r, pl.ds(j * _LANES, _LANES)]
                    d = f - c
                    new.append(accs[j] + w * d * d)
                return tuple(new)

            acc = lax.fori_loop(0, _CHUNK, row_body, acc)
        total = acc[0]
        for j in range(1, vecs_per_row):
            total = total + acc[j]
        acc_v[...] = total
        pltpu.sync_copy(acc_v, out_hbm.at[pl.ds(wid * _LANES, _LANES)])

    return sc_kernel


def kernel(feat, A, label, centers):
    B, D = feat.shape
    partials = _build(B, D)(feat, A, label.astype(jnp.int32), centers)
    return jnp.sum(partials) * (0.5 / B)


# speedup vs baseline: 1.1877x; 1.1877x over previous
import functools

import jax
import jax.numpy as jnp
from jax import lax
from jax.experimental import pallas as pl
from jax.experimental.pallas import tpu as pltpu
from jax.experimental.pallas import tpu_sc as plsc

_NUM_CORES = 2
_NUM_SUBCORES = 16
_NW = _NUM_CORES * _NUM_SUBCORES
_LANES = 16
_CHUNK = 128


@functools.cache
def _build(B, D):
    rows_per_w = B // _NW
    n_chunks = rows_per_w // _CHUNK
    vecs_per_row = D // _LANES
    assert rows_per_w * _NW == B and n_chunks * _CHUNK == rows_per_w
    assert vecs_per_row * _LANES == D

    mesh = plsc.VectorSubcoreMesh(core_axis_name="c", subcore_axis_name="s")

    @functools.partial(
        pl.kernel,
        out_type=jax.ShapeDtypeStruct((_NW * _LANES,), jnp.float32),
        mesh=mesh,
        scratch_types=[
            pltpu.VMEM((n_chunks, _CHUNK), jnp.int32),
            pltpu.VMEM((2, _CHUNK, D), jnp.float32),
            pltpu.VMEM((2, _CHUNK, D), jnp.float32),
            pltpu.VMEM((2, _CHUNK, D), jnp.float32),
            pltpu.VMEM((_LANES,), jnp.float32),
            [pltpu.SemaphoreType.DMA] * 6,
        ],
    )
    def sc_kernel(feat_hbm, a_hbm, label_hbm, centers_hbm, out_hbm,
                  idx_v, cent_v, feat_v, a_v, acc_v, sems):
        wid = lax.axis_index("s") * _NUM_CORES + lax.axis_index("c")
        base = wid * rows_per_w
        pltpu.sync_copy(label_hbm.at[pl.ds(wid * n_chunks, n_chunks), :],
                        idx_v)

        def fire(ci, slot):
            row0 = base + ci * _CHUNK
            return (
                pltpu.async_copy(centers_hbm.at[idx_v.at[ci]],
                                 cent_v.at[slot], sems[3 * slot]),
                pltpu.async_copy(feat_hbm.at[pl.ds(row0, _CHUNK), :],
                                 feat_v.at[slot], sems[3 * slot + 1]),
                pltpu.async_copy(a_hbm.at[pl.ds(row0, _CHUNK), :],
                                 a_v.at[slot], sems[3 * slot + 2]),
            )

        acc = tuple(jnp.zeros((_LANES,), jnp.float32)
                    for _ in range(vecs_per_row))
        in_flight = fire(0, 0)
        for ci in range(n_chunks):
            slot = ci % 2
            cur = in_flight
            if ci + 1 < n_chunks:
                in_flight = fire(ci + 1, 1 - slot)
            for c in cur:
                c.wait()

            def row_body(r, accs):
                new = []
                for j in range(vecs_per_row):
                    f = feat_v[slot, r, pl.ds(j * _LANES, _LANES)]
                    c = cent_v[slot, r, pl.ds(j * _LANES, _LANES)]
                    w = a_v[slot, r, pl.ds(j * _LANES, _LANES)]
                    d = f - c
                    new.append(accs[j] + w * d * d)
                return tuple(new)

            acc = lax.fori_loop(0, _CHUNK, row_body, acc)
        total = acc[0]
        for j in range(1, vecs_per_row):
            total = total + acc[j]
        acc_v[...] = total
        pltpu.sync_copy(acc_v, out_hbm.at[pl.ds(wid * _LANES, _LANES)])

    return sc_kernel


def kernel(feat, A, label, centers):
    B, D = feat.shape
    label2d = label.astype(jnp.int32).reshape(B // _CHUNK, _CHUNK)
    partials = _build(B, D)(feat, A, label2d, centers)
    return jnp.sum(partials) * (0.5 / B)
